# Initial kernel scaffold; baseline (speedup 1.0000x reference)
import jax
import jax.numpy as jnp
from jax.experimental import pallas as pl


def kernel(x, weights_row, weights_column):
    raise NotImplementedError("write your pallas kernel here")


# speedup vs baseline: 5.9240x; 5.9240x over previous
import functools

import jax
import jax.numpy as jnp
from jax import lax
from jax.experimental import pallas as pl
from jax.experimental.pallas import tpu as pltpu
from jax.experimental.pallas import tpu_sc as plsc

B, H, W = 64, 512, 512
SIZE = H * W
CH = 16384
NCH = SIZE // CH
L = 16

_NC = 2
_NS = 16
_NW = _NC * _NS
_RPW = B // _NW


def _make_roll_kernel():
    mesh = plsc.VectorSubcoreMesh(core_axis_name="c", subcore_axis_name="s")

    @functools.partial(
        pl.kernel,
        out_type=jax.ShapeDtypeStruct((B * SIZE,), jnp.float32),
        mesh=mesh,
        scratch_types=[
            pltpu.MemorySpace.VMEM((L,), jnp.int32),
            pltpu.MemorySpace.VMEM((2 * CH,), jnp.float32),
            pltpu.MemorySpace.VMEM((CH,), jnp.float32),
        ],
    )
    def roll_k(x_hbm, svec_hbm, out_hbm, svec_v, buf, obuf):
        wid = lax.axis_index("s") * _NC + lax.axis_index("c")
        pltpu.sync_copy(svec_hbm, svec_v)
        s = svec_v[...][0]

        for r in range(_RPW):
            b = wid * _RPW + r
            row0 = b * SIZE
            for c in range(NCH):
                o0 = c * CH
                g = lax.rem(o0 - s + SIZE, SIZE)
                start1 = (g // CH) * CH
                start2 = lax.rem(start1 + CH, SIZE)
                d = g - start1
                pltpu.sync_copy(x_hbm.at[pl.ds(row0 + start1, CH)],
                                buf.at[pl.ds(0, CH)])
                pltpu.sync_copy(x_hbm.at[pl.ds(row0 + start2, CH)],
                                buf.at[pl.ds(CH, CH)])

                def shift_copy(j, carry):
                    obuf[pl.ds(j * L, L)] = buf[pl.ds(d + j * L, L)]
                    return carry

                lax.fori_loop(0, CH // L, shift_copy, 0)
                pltpu.sync_copy(obuf, out_hbm.at[pl.ds(row0 + o0, CH)])

    return roll_k


_roll = _make_roll_kernel()


@jax.jit
def kernel(x, weights_row, weights_column):
    b, h, w = x.shape
    size = h * w
    shift = jnp.round(weights_row + h * weights_column).astype(jnp.int32)
    s = jnp.mod(shift, size).astype(jnp.int32)
    svec = jnp.where(jnp.arange(L) == 0, s, 0).astype(jnp.int32)
    out = _roll(x.reshape(-1), svec)
    return out.reshape(b, h, w)
